# baseline (speedup 1.0000x reference)
import functools

import numpy as np
import jax
import jax.numpy as jnp
from jax import lax
from jax.experimental import pallas as pl
from jax.experimental.pallas import tpu as pltpu
from jax.experimental.pallas import tpu_sc as plsc

N = 10000
E = 320000
D = 128
H = 128
C = 16
G = 64
ALPHA = 0.2
BN_EPS = 1e-5

NTILES = 16
NB = 216
B = 96
EPT = NB * B
EP = NTILES * EPT
WROW = 144
HALF = 2048
ACCR = HALF + 128
RPT = HALF // 16
NPH = 5
NPAD = NPH * HALF
NBLK = 5
BLK = N // NBLK
NBUF = 2

def _combine(a_ref):
    s = a_ref[...]
    hp = s[:, :H]
    rs = s[:, H:H + 1]
    x = hp / (rs + 1e-16)
    return jnp.where(x > 0, x, jnp.exp(x) - 1.0)


def _tc_matmul(x_or_acc, wp, cp, ws, cs, combine):
    def body(a_ref, wp_ref, cp_ref, ws_ref, cs_ref, o1_ref, o2_ref):
        x = _combine(a_ref) if combine else a_ref[...]
        h = jnp.dot(x, wp_ref[...], preferred_element_type=jnp.float32)
        o1_ref[...] = h + cp_ref[...]
        o2_ref[...] = jnp.dot(x, ws_ref[...],
                              preferred_element_type=jnp.float32) + cs_ref[...]

    in_w = WROW if combine else H
    return pl.pallas_call(
        body,
        grid=(NBLK,),
        in_specs=[pl.BlockSpec((BLK, in_w), lambda i: (i, 0)),
                  pl.BlockSpec((H, H), lambda i: (0, 0)),
                  pl.BlockSpec((1, H), lambda i: (0, 0)),
                  pl.BlockSpec((H, 8), lambda i: (0, 0)),
                  pl.BlockSpec((1, 8), lambda i: (0, 0))],
        out_specs=[pl.BlockSpec((BLK, H), lambda i: (i, 0)),
                   pl.BlockSpec((BLK, 8), lambda i: (i, 0))],
        out_shape=[jax.ShapeDtypeStruct((N, H), jnp.float32),
                   jax.ShapeDtypeStruct((N, 8), jnp.float32)],
    )(x_or_acc, wp, cp, ws, cs)


def _tc_pool_mlp(acc, gi3, fc1w, fc1b, fc2w, fc2b, fc3w, fc3b):
    def body(a_ref, gi_ref, w1_ref, b1_ref, w2_ref, b2_ref, w3_ref, b3_ref,
             o_ref, p_ref):
        i = pl.program_id(0)
        x = _combine(a_ref)
        gi = gi_ref[0, 0, :]
        oh = (gi[:, None] == lax.broadcasted_iota(jnp.int32, (BLK, G), 1)
              ).astype(jnp.float32)
        contrib = lax.dot_general(oh, x, (((0,), (0,)), ((), ())),
                                  preferred_element_type=jnp.float32)

        @pl.when(i == 0)
        def _():
            p_ref[...] = contrib

        @pl.when(i > 0)
        def _():
            p_ref[...] += contrib

        @pl.when(i == NBLK - 1)
        def _():
            p = p_ref[...]
            z = jnp.maximum(jnp.dot(p, w1_ref[...],
                                    preferred_element_type=jnp.float32)
                            + b1_ref[...], 0.0)
            z = jnp.maximum(jnp.dot(z, w2_ref[...],
                                    preferred_element_type=jnp.float32)
                            + b2_ref[...], 0.0)
            o_ref[...] = jnp.dot(z, w3_ref[...],
                                 preferred_element_type=jnp.float32) + b3_ref[...]

    return pl.pallas_call(
        body,
        grid=(NBLK,),
        in_specs=[pl.BlockSpec((BLK, WROW), lambda i: (i, 0)),
                  pl.BlockSpec((1, 1, BLK), lambda i: (i, 0, 0)),
                  pl.BlockSpec((H, 2 * H), lambda i: (0, 0)),
                  pl.BlockSpec((1, 2 * H), lambda i: (0, 0)),
                  pl.BlockSpec((2 * H, H), lambda i: (0, 0)),
                  pl.BlockSpec((1, H), lambda i: (0, 0)),
                  pl.BlockSpec((H, C), lambda i: (0, 0)),
                  pl.BlockSpec((1, C), lambda i: (0, 0))],
        out_specs=pl.BlockSpec((G, C), lambda i: (0, 0)),
        out_shape=jax.ShapeDtypeStruct((G, C), jnp.float32),
        scratch_shapes=[pltpu.VMEM((G, H), jnp.float32)],
    )(acc, gi3, fc1w, fc1b.reshape(1, -1), fc2w, fc2b.reshape(1, -1),
      fc3w, fc3b.reshape(1, -1))


def _sc_edge(hb, s1, s2, encp, zrows):
    mesh = plsc.VectorSubcoreMesh(core_axis_name="c", subcore_axis_name="s",
                                  num_cores=1)

    @functools.partial(
        pl.kernel,
        out_type=jax.ShapeDtypeStruct((NPAD, WROW), jnp.float32),
        mesh=mesh,
        compiler_params=pltpu.CompilerParams(needs_layout_passes=False,
                                             use_tc_tiling_on_sc=False),
        scratch_types=[
            pltpu.VMEM((NB, B), jnp.int32),
            pltpu.VMEM((N,), jnp.float32),
            pltpu.VMEM((N,), jnp.float32),
            pltpu.VMEM((NBUF * B,), jnp.float32),
            pltpu.VMEM((NBUF, B), jnp.int32),
            pltpu.VMEM((NBUF, B), jnp.int32),
            pltpu.VMEM((NBUF, B, H), jnp.float32),
            pltpu.VMEM((NBUF, B, WROW), jnp.float32),
            pltpu.VMEM_SHARED((ACCR, WROW), jnp.float32),
            pltpu.SemaphoreType.DMA((NBUF,)),
            pltpu.SemaphoreType.DMA((NBUF,)),
        ],
    )
    def k(hb_hbm, s1_hbm, s2_hbm, enc_hbm, z_hbm, out_hbm,
          enc_v, s1_v, s2_v, e_v, sidx_v, didx_v, gb_v, sb_v, acc_sp,
          gsem, ssem):
        sid = lax.axis_index("s")

        pltpu.sync_copy(enc_hbm.at[sid], enc_v)
        pltpu.sync_copy(s1_hbm, s1_v)
        pltpu.sync_copy(s2_hbm, s2_v)

        zbase = sid * RPT
        lim = jnp.maximum(jnp.minimum(E - sid * EPT, EPT), 0)
        iota = lax.broadcasted_iota(jnp.int32, (16,), 0)
        onehot0 = jnp.where(iota == 0, 1.0, 0.0)
        zeros16 = jnp.zeros((16,), jnp.int32)

        def decode_didx(b, k_):
            for g in range(B // 16):
                sl = pl.ds(g * 16, 16)
                didx_v[k_, sl] = enc_v[b, sl] & 16383

        def issue_g(b, k_):
            decode_didx(b, k_)
            pltpu.async_copy(hb_hbm.at[didx_v.at[k_]], gb_v.at[k_],
                             gsem.at[k_])

        def wait_g(k_):
            pltpu.make_async_copy(hb_hbm.at[didx_v.at[k_]], gb_v.at[k_],
                                  gsem.at[k_]).wait()

        def issue_s(k_):
            pltpu.async_copy(sb_v.at[k_], acc_sp.at[sidx_v.at[k_]],
                             ssem.at[k_], add=True)

        def wait_s(k_):
            pltpu.make_async_copy(sb_v.at[k_], acc_sp.at[sidx_v.at[k_]],
                                  ssem.at[k_]).wait()

        def scale(k_):
            def sbody(j, carry):
                for u in range(2):
                    ii = j * 2 + u
                    ebc = plsc.load_gather(e_v, [zeros16 + (k_ * B + ii)])
                    for g in range(8):
                        sl = pl.ds(g * 16, 16)
                        sb_v[k_, ii, sl] = ebc * gb_v[k_, ii, sl]
                    sb_v[k_, ii, pl.ds(H, 16)] = ebc * onehot0
                return carry

            lax.fori_loop(0, B // 2, sbody, 0)

        for p in range(NPH):
            pbase = p * HALF

            pltpu.sync_copy(z_hbm, acc_sp.at[pl.ds(zbase, RPT)])
            plsc.subcore_barrier()

            def compute_e(b, k_):
                for g in range(B // 16):
                    sl = pl.ds(g * 16, 16)
                    p16 = enc_v[b, sl]
                    si = p16 >> 14
                    di = p16 & 16383
                    own = (si >= pbase) & (si < pbase + HALF)
                    sidx_v[k_, sl] = jnp.where(own, si - pbase, HALF)
                    v1 = plsc.load_gather(s1_v, [si])
                    v2 = plsc.load_gather(s2_v, [di])
                    lg = v1 + v2
                    lrelu = jnp.maximum(lg, ALPHA * lg)
                    ev = jnp.exp(-lrelu)
                    loc = iota + (b * B + g * 16)
                    ev = jnp.where((loc < lim) & own, ev, 0.0)
                    e_v[pl.ds(k_ * B + g * 16, 16)] = ev

            for k_ in range(NBUF):
                issue_g(k_, k_)

            def mbody(r, carry):
                for k_ in range(NBUF):
                    b = r * NBUF + k_

                    @pl.when(r > 0)
                    def _():
                        wait_s(k_)

                    compute_e(b, k_)
                    wait_g(k_)
                    scale(k_)
                    issue_s(k_)

                    @pl.when(r < NB // NBUF - 1)
                    def _():
                        issue_g(b + NBUF, k_)

                return carry

            lax.fori_loop(0, NB // NBUF, mbody, 0)
            for k_ in range(NBUF):
                wait_s(k_)

            plsc.subcore_barrier()
            pltpu.sync_copy(acc_sp.at[pl.ds(zbase, RPT)],
                            out_hbm.at[pl.ds(pbase + zbase, RPT)])

    return k(hb, s1, s2, encp, zrows)


def _fold_weights(W, a, g=None, b=None):
    f32 = jnp.float32
    aL = a[0, :H]
    aR = a[0, H:]
    if g is None:
        Wf = W
        c = jnp.zeros((H,), f32)
    else:
        gamma = g / jnp.sqrt(1.0 + BN_EPS)
        Wf = gamma[:, None] * W
        c = b @ W
    wp = Wf
    cp = c.reshape(1, H)
    ws = jnp.concatenate([(Wf @ aL)[:, None], (Wf @ aR)[:, None],
                          jnp.zeros((H, 6), f32)], axis=1)
    cs = jnp.concatenate([(c @ aL)[None], (c @ aR)[None],
                          jnp.zeros((6,), f32)]).reshape(1, 8)
    return wp, cp, ws, cs


def kernel(adjacency, input_feature, graph_indicator, labels, W1, a1, W2, a2,
           W3, a3, bn2_g, bn2_b, bn3_g, bn3_b, fc1_W, fc1_b, fc2_W, fc2_b,
           fc3_W, fc3_b):
    pad = EP - E
    enc = adjacency[0] * 16384 + adjacency[1]
    encp = jnp.concatenate(
        [enc, jnp.zeros((pad,), jnp.int32)]).reshape(NTILES, NB, B)
    zrows = jnp.zeros((RPT, WROW), jnp.float32)
    gi3 = graph_indicator.reshape(NBLK, 1, BLK)

    p1 = _fold_weights(W1, a1)
    p2 = _fold_weights(W2, a2, bn2_g, bn2_b)
    p3 = _fold_weights(W3, a3, bn3_g, bn3_b)

    def edge_stage(hb, hs):
        return _sc_edge(hb, hs[:, 0], hs[:, 1], encp, zrows)

    hb, hs = _tc_matmul(input_feature, *p1, combine=False)
    acc = edge_stage(hb, hs)
    hb, hs = _tc_matmul(acc, *p2, combine=True)
    acc = edge_stage(hb, hs)
    hb, hs = _tc_matmul(acc, *p3, combine=True)
    acc = edge_stage(hb, hs)
    return _tc_pool_mlp(acc, gi3, fc1_W, fc1_b, fc2_W, fc2_b, fc3_W, fc3_b)
